# Initial kernel scaffold; baseline (speedup 1.0000x reference)
import jax
import jax.numpy as jnp
from jax.experimental import pallas as pl


def kernel(x, edge_index, edge_index_b, lam, id_new_value_old, W0, b0, W1, b1, Wlin, blin):
    raise NotImplementedError("write your pallas kernel here")


# speedup vs baseline: 14.4075x; 14.4075x over previous
import functools

import jax
import jax.numpy as jnp
from jax import lax
from jax.experimental import pallas as pl
from jax.experimental.pallas import tpu as pltpu
from jax.experimental.pallas import tpu_sc as plsc

NC = 2
NS = 16
NW = NC * NS
K = 125
KP = 80
DW = 16
NP = 10240
RPT = NP // NS
ZR = 32
RB = 1024


def _sc_deg_perm(A, perm1, dsta2, dstb2, ones1, zeros1):
    d = A.shape[1]
    cpt = dsta2.shape[0] // NW
    pch = perm1.shape[0] // KP

    mesh = plsc.VectorSubcoreMesh(core_axis_name="c", subcore_axis_name="s")

    @functools.partial(
        pl.kernel,
        mesh=mesh,
        out_type=[
            jax.ShapeDtypeStruct((NP, d), jnp.float32),
            jax.ShapeDtypeStruct((NC, NP), jnp.float32),
            jax.ShapeDtypeStruct((NC, NP), jnp.float32),
        ],
        scratch_types=[
            pltpu.VMEM((cpt, K), jnp.int32),
            pltpu.VMEM((cpt, K), jnp.int32),
            pltpu.VMEM((KP,), jnp.int32),
            pltpu.VMEM((KP, d), jnp.float32),
            pltpu.VMEM((K,), jnp.float32),
            pltpu.VMEM_SHARED((NP,), jnp.float32),
            pltpu.VMEM_SHARED((NP,), jnp.float32),
        ],
    )
    def body(a_hbm, perm_hbm, da_hbm, db_hbm, ones_hbm, z_hbm,
             aperm_hbm, dap_hbm, dbp_hbm,
             ia, ib, pidx, rows, ones, acc_a, acc_b):
        cid = lax.axis_index("c")
        sid = lax.axis_index("s")
        wid = sid * NC + cid

        pltpu.sync_copy(da_hbm.at[pl.ds(wid * cpt, cpt)], ia)
        pltpu.sync_copy(db_hbm.at[pl.ds(wid * cpt, cpt)], ib)
        pltpu.sync_copy(ones_hbm, ones)
        pltpu.sync_copy(z_hbm, acc_a.at[pl.ds(sid * RPT, RPT)])
        pltpu.sync_copy(z_hbm, acc_b.at[pl.ds(sid * RPT, RPT)])
        plsc.subcore_barrier()

        def deg_chunk(j, _):
            pltpu.sync_copy(ones, acc_a.at[ia.at[j]], add=True)
            pltpu.sync_copy(ones, acc_b.at[ib.at[j]], add=True)
            return 0
        lax.fori_loop(0, cpt, deg_chunk, 0)

        nloop = (pch + NW - 1) // NW

        def perm_chunk(j, _):
            c = wid + j * NW

            @pl.when(c < pch)
            def _():
                pltpu.sync_copy(perm_hbm.at[pl.ds(c * KP, KP)], pidx)
                pltpu.sync_copy(a_hbm.at[pidx], rows)
                pltpu.sync_copy(rows, aperm_hbm.at[pl.ds(c * KP, KP)])
            return 0
        lax.fori_loop(0, nloop, perm_chunk, 0)

        plsc.subcore_barrier()
        pltpu.sync_copy(acc_a.at[pl.ds(sid * RPT, RPT)],
                        dap_hbm.at[cid, pl.ds(sid * RPT, RPT)])
        pltpu.sync_copy(acc_b.at[pl.ds(sid * RPT, RPT)],
                        dbp_hbm.at[cid, pl.ds(sid * RPT, RPT)])

    return body(A, perm1, dsta2, dstb2, ones1, zeros1)


def _sc_segment(tables, which, srca2, dsta2, srcb2, dstb2, zrows,
                perm1=None, gsrc=None):
    d = tables[0].shape[1]
    cpt = srca2.shape[0] // NW
    nt = len(tables)
    do_perm = perm1 is not None
    pch = perm1.shape[0] // KP if do_perm else 0

    mesh = plsc.VectorSubcoreMesh(core_axis_name="c", subcore_axis_name="s")

    out_type = [jax.ShapeDtypeStruct((NC, NP, d), jnp.float32)
                for _ in range(nt)]
    if do_perm:
        out_type.append(jax.ShapeDtypeStruct((NP, d), jnp.float32))

    scratch = [
        pltpu.VMEM((cpt, K), jnp.int32),
        pltpu.VMEM((cpt, K), jnp.int32),
        pltpu.VMEM((KP,), jnp.int32),
        pltpu.VMEM((KP, d), jnp.float32),
        pltpu.VMEM((K, d), jnp.float32),
        pltpu.VMEM_SHARED((NP, d), jnp.float32),
    ]

    @functools.partial(pl.kernel, mesh=mesh, out_type=out_type,
                       scratch_types=scratch)
    def body(*refs):
        y_hbm = refs[:nt]
        src_hbm = {0: refs[nt], 1: refs[nt + 2]}
        dst_hbm = {0: refs[nt + 1], 1: refs[nt + 3]}
        z_hbm = refs[nt + 4]
        pos = nt + 5
        if do_perm:
            gsrc_hbm = refs[pos]
            perm_hbm = refs[pos + 1]
            pos += 2
        outs = refs[pos:pos + nt]
        pos += nt
        if do_perm:
            gout_hbm = refs[pos]
            pos += 1
        i_s, i_d, pidx, prows, rows, acc = refs[pos:pos + 6]

        cid = lax.axis_index("c")
        sid = lax.axis_index("s")
        wid = sid * NC + cid

        if do_perm:
            nloop = (pch + NW - 1) // NW

            def perm_chunk(j, _):
                c = wid + j * NW

                @pl.when(c < pch)
                def _():
                    pltpu.sync_copy(perm_hbm.at[pl.ds(c * KP, KP)], pidx)
                    pltpu.sync_copy(gsrc_hbm.at[pidx], prows)
                    pltpu.sync_copy(prows, gout_hbm.at[pl.ds(c * KP, KP)])
                return 0
            lax.fori_loop(0, nloop, perm_chunk, 0)

        staged = None
        for t in range(nt):
            if staged != which[t]:
                pltpu.sync_copy(src_hbm[which[t]].at[pl.ds(wid * cpt, cpt)],
                                i_s)
                pltpu.sync_copy(dst_hbm[which[t]].at[pl.ds(wid * cpt, cpt)],
                                i_d)
                staged = which[t]
            pltpu.sync_copy(z_hbm, acc.at[pl.ds(sid * RPT, RPT)])
            plsc.subcore_barrier()

            def chunk(j, _):
                pltpu.sync_copy(y_hbm[t].at[i_s.at[j]], rows)
                pltpu.sync_copy(rows, acc.at[i_d.at[j]], add=True)
                return 0
            lax.fori_loop(0, cpt, chunk, 0)

            plsc.subcore_barrier()
            pltpu.sync_copy(acc.at[pl.ds(sid * RPT, RPT)],
                            outs[t].at[cid, pl.ds(sid * RPT, RPT)])

    args = list(tables) + [srca2, dsta2, srcb2, dstb2, zrows]
    if do_perm:
        args += [gsrc, perm1]
    return body(*args)


def _t_matmul(x, w):
    n, d = x.shape
    f = w.shape[1]

    def body(x_ref, w_ref, o_ref):
        o_ref[...] = jnp.dot(x_ref[...], w_ref[...],
                             preferred_element_type=jnp.float32)

    return pl.pallas_call(
        body,
        grid=(n // RB,),
        in_specs=[pl.BlockSpec((RB, d), lambda i: (i, 0)),
                  pl.BlockSpec((d, f), lambda i: (0, 0))],
        out_specs=pl.BlockSpec((RB, f), lambda i: (i, 0)),
        out_shape=jax.ShapeDtypeStruct((n, f), jnp.float32),
    )(x, w)


def _deg_from(p_ref):
    return p_ref[0, :] + p_ref[1, :] + 1.0


def _t_build_tables(A, Aperm, dap, dbp, lam_arr):
    n, d = A.shape

    def body(a_ref, ap_ref, da_ref, db_ref, lam_ref,
             y1_ref, y2_ref, y3_ref):
        lam = lam_ref[0, 0]
        dinv_a = lax.rsqrt(_deg_from(da_ref))[:, None]
        dinv_b = lax.rsqrt(_deg_from(db_ref))[:, None]
        a = a_ref[...]
        m = lam * a + (1.0 - lam) * ap_ref[...]
        y1_ref[...] = m * dinv_a
        y2_ref[...] = m * dinv_b
        y3_ref[...] = a * dinv_a

    bs = pl.BlockSpec((RB, d), lambda i: (i, 0))
    ds_ = pl.BlockSpec((NC, RB), lambda i: (0, i))
    return pl.pallas_call(
        body,
        grid=(n // RB,),
        in_specs=[bs, bs, ds_, ds_,
                  pl.BlockSpec(memory_space=pltpu.SMEM)],
        out_specs=[bs, bs, bs],
        out_shape=[jax.ShapeDtypeStruct((n, d), jnp.float32)] * 3,
    )(A, Aperm, dap, dbp, lam_arr)


def _t_mid(s1p, s2p, s3p, A, Aperm, dap, dbp, b0, W1, lam_arr):
    n, d = A.shape
    h = W1.shape[1]

    def body(s1_ref, s2_ref, s3_ref, a_ref, ap_ref, da_ref, db_ref,
             b0_ref, w1_ref, lam_ref, p_ref, y4_ref, y5_ref):
        lam = lam_ref[0, 0]
        deg_a = _deg_from(da_ref)
        deg_b = _deg_from(db_ref)
        dinv_a = lax.rsqrt(deg_a)[:, None]
        dinv_b = lax.rsqrt(deg_b)[:, None]
        ideg_a = (1.0 / deg_a)[:, None]
        ideg_b = (1.0 / deg_b)[:, None]
        a = a_ref[...]
        b0v = b0_ref[0, :][None, :]
        s1 = s1_ref[0] + s1_ref[1]
        s2 = s2_ref[0] + s2_ref[1]
        s3 = s3_ref[0] + s3_ref[1]
        hh = jnp.maximum(dinv_a * s1 + a * ideg_a + b0v, 0.0)
        hb = jnp.maximum(dinv_b * s2 + ap_ref[...] * ideg_b + b0v, 0.0)
        xn = jnp.maximum(dinv_a * s3 + a * ideg_a + b0v, 0.0)
        w1 = w1_ref[...]
        p = jnp.dot(xn, w1, preferred_element_type=jnp.float32)
        q = jnp.dot(lam * hh + (1.0 - lam) * hb, w1,
                    preferred_element_type=jnp.float32)
        p_ref[...] = p
        y4_ref[...] = q * dinv_a
        y5_ref[...] = q * dinv_b

    bs = pl.BlockSpec((RB, d), lambda i: (i, 0))
    ps = pl.BlockSpec((NC, RB, d), lambda i: (0, i, 0))
    ds_ = pl.BlockSpec((NC, RB), lambda i: (0, i))
    return pl.pallas_call(
        body,
        grid=(n // RB,),
        in_specs=[ps, ps, ps, bs, bs, ds_, ds_,
                  pl.BlockSpec((1, d), lambda i: (0, 0)),
                  pl.BlockSpec((d, h), lambda i: (0, 0)),
                  pl.BlockSpec(memory_space=pltpu.SMEM)],
        out_specs=[bs, bs, bs],
        out_shape=[jax.ShapeDtypeStruct((n, h), jnp.float32)] * 3,
    )(s1p, s2p, s3p, A, Aperm, dap, dbp, b0, W1, lam_arr)


def _t_final(s4p, s5p, P, Pperm, dap, dbp, b1, Wlin, blin, lam_arr):
    n, h = P.shape
    c = Wlin.shape[1]

    def body(s4_ref, s5_ref, p_ref, pp_ref, da_ref, db_ref,
             b1_ref, wl_ref, bl_ref, lam_ref, o_ref):
        lam = lam_ref[0, 0]
        deg_a = _deg_from(da_ref)
        deg_b = _deg_from(db_ref)
        dinv_a = lax.rsqrt(deg_a)[:, None]
        dinv_b = lax.rsqrt(deg_b)[:, None]
        ideg_a = (1.0 / deg_a)[:, None]
        ideg_b = (1.0 / deg_b)[:, None]
        b1v = b1_ref[0, :][None, :]
        s4 = s4_ref[0] + s4_ref[1]
        s5 = s5_ref[0] + s5_ref[1]
        xf = jnp.maximum(dinv_a * s4 + p_ref[...] * ideg_a + b1v, 0.0)
        xfb = jnp.maximum(dinv_b * s5 + pp_ref[...] * ideg_b + b1v, 0.0)
        xo = lam * xf + (1.0 - lam) * xfb
        logits = jnp.dot(xo, wl_ref[...],
                         preferred_element_type=jnp.float32) + bl_ref[0, :][None, :]
        zmax = jnp.max(logits, axis=-1, keepdims=True)
        z = logits - zmax
        lse = jnp.log(jnp.sum(jnp.exp(z), axis=-1, keepdims=True))
        o_ref[...] = z - lse

    bs = pl.BlockSpec((RB, h), lambda i: (i, 0))
    ps = pl.BlockSpec((NC, RB, h), lambda i: (0, i, 0))
    ds_ = pl.BlockSpec((NC, RB), lambda i: (0, i))
    return pl.pallas_call(
        body,
        grid=(n // RB,),
        in_specs=[ps, ps, bs, bs, ds_, ds_,
                  pl.BlockSpec((1, h), lambda i: (0, 0)),
                  pl.BlockSpec((h, c), lambda i: (0, 0)),
                  pl.BlockSpec((1, c), lambda i: (0, 0)),
                  pl.BlockSpec(memory_space=pltpu.SMEM)],
        out_specs=pl.BlockSpec((RB, c), lambda i: (i, 0)),
        out_shape=jax.ShapeDtypeStruct((n, c), jnp.float32),
    )(s4p, s5p, P, Pperm, dap, dbp, b1, Wlin, blin, lam_arr)


def kernel(x, edge_index, edge_index_b, lam, id_new_value_old,
           W0, b0, W1, b1, Wlin, blin):
    n, d = x.shape
    e = edge_index.shape[1]
    assert e % (NW * K) == 0 and n <= NP and NP % KP == 0

    lam_arr = jnp.reshape(jnp.asarray(lam, jnp.float32), (1, 1))
    srca2 = edge_index[0].reshape(e // K, K)
    dsta2 = edge_index[1].reshape(e // K, K)
    srcb2 = edge_index_b[0].reshape(e // K, K)
    dstb2 = edge_index_b[1].reshape(e // K, K)
    perm1 = jnp.pad(id_new_value_old, (0, NP - n))
    xp = jnp.pad(x, ((0, NP - n), (0, 0)))
    b0r = b0.reshape(1, -1)
    b1r = b1.reshape(1, -1)
    blinr = blin.reshape(1, -1)
    ones1 = jnp.ones((K,), jnp.float32)
    zeros1 = jnp.zeros((RPT,), jnp.float32)
    zrows = jnp.zeros((RPT, x.shape[1]), jnp.float32)

    A = _t_matmul(xp, W0)
    Aperm, dap, dbp = _sc_deg_perm(A, perm1, dsta2, dstb2, ones1, zeros1)
    y1, y2, y3 = _t_build_tables(A, Aperm, dap, dbp, lam_arr)
    s1p, s3p, s2p = _sc_segment([y1, y3, y2], [0, 0, 1],
                                srca2, dsta2, srcb2, dstb2, zrows)
    P, y4, y5 = _t_mid(s1p, s2p, s3p, A, Aperm, dap, dbp, b0r, W1, lam_arr)
    s4p, s5p, Pperm = _sc_segment([y4, y5], [0, 1],
                                  srca2, dsta2, srcb2, dstb2, zrows,
                                  perm1=perm1, gsrc=P)
    out = _t_final(s4p, s5p, P, Pperm, dap, dbp, b1r, Wlin, blinr, lam_arr)
    return out[:n]
